# baseline (speedup 1.0000x reference)
import functools

import jax
import jax.numpy as jnp
from jax import lax
from jax.experimental import pallas as pl
from jax.experimental.pallas import tpu as pltpu
from jax.experimental.pallas import tpu_sc as plsc

NC = 2
NS = 16
L = 16
NW = NC * NS

BATCH = 16384
EMBED = 64
CHUNK = 128
B_PER_W = BATCH // NW
N_CHUNKS = B_PER_W // CHUNK
GROUPS = B_PER_W // L


def _mf_body(u_id_hbm, i_id_hbm, user_emb_hbm, user_bias_hbm,
             item_emb_hbm, item_bias_hbm, mean_hbm, out_hbm,
             uidx_v, iidx_v, urows_v, irows_v, ub_v, ib_v, out_v, mean_v,
             sem):
    wid = lax.axis_index("s") * NC + lax.axis_index("c")
    row0 = wid * N_CHUNKS

    pltpu.sync_copy(u_id_hbm.at[pl.ds(row0, N_CHUNKS)], uidx_v)
    pltpu.sync_copy(i_id_hbm.at[pl.ds(row0, N_CHUNKS)], iidx_v)
    pltpu.sync_copy(mean_hbm, mean_v)

    copies = []
    for c in range(N_CHUNKS):
        sl = pl.ds(c * CHUNK, CHUNK)
        copies.append(pltpu.async_copy(
            user_emb_hbm.at[uidx_v.at[c]], urows_v.at[sl], sem))
        copies.append(pltpu.async_copy(
            item_emb_hbm.at[iidx_v.at[c]], irows_v.at[sl], sem))
        copies.append(pltpu.async_copy(
            user_bias_hbm.at[uidx_v.at[c]], ub_v.at[sl], sem))
        copies.append(pltpu.async_copy(
            item_bias_hbm.at[iidx_v.at[c]], ib_v.at[sl], sem))
    for cp in copies:
        cp.wait()

    mean_vec = mean_v[...]
    zeros = jnp.zeros((L,), jnp.int32)

    def group_body(g, _):
        rows = g * L + lax.iota(jnp.int32, L)
        acc = ub_v[pl.ds(g * L, L)] + ib_v[pl.ds(g * L, L)] + mean_vec
        for j in range(EMBED):
            col = jnp.full((L,), j, jnp.int32)
            uj = plsc.load_gather(urows_v, [rows, col])
            ij = plsc.load_gather(irows_v, [rows, col])
            acc = acc + uj * ij
        out_v[pl.ds(g * L, L)] = acc
        return 0

    lax.fori_loop(0, GROUPS, group_body, 0)

    pltpu.sync_copy(out_v, out_hbm.at[pl.ds(wid * B_PER_W, B_PER_W)])


@functools.partial(jax.jit, static_argnames=())
def kernel(u_id, i_id, user_emb, user_bias, item_emb, item_bias, mean):
    mesh = plsc.VectorSubcoreMesh(
        core_axis_name="c", subcore_axis_name="s",
        num_cores=NC, num_subcores=NS)
    f = pl.kernel(
        _mf_body,
        out_type=jax.ShapeDtypeStruct((BATCH,), jnp.float32),
        mesh=mesh,
        compiler_params=pltpu.CompilerParams(
            needs_layout_passes=False, use_tc_tiling_on_sc=False),
        scratch_types=[
            pltpu.VMEM((N_CHUNKS, CHUNK), jnp.int32),
            pltpu.VMEM((N_CHUNKS, CHUNK), jnp.int32),
            pltpu.VMEM((B_PER_W, EMBED), jnp.float32),
            pltpu.VMEM((B_PER_W, EMBED), jnp.float32),
            pltpu.VMEM((B_PER_W,), jnp.float32),
            pltpu.VMEM((B_PER_W,), jnp.float32),
            pltpu.VMEM((B_PER_W,), jnp.float32),
            pltpu.VMEM((L,), jnp.float32),
            pltpu.SemaphoreType.DMA,
        ],
    )
    u2 = u_id.reshape(BATCH // CHUNK, CHUNK).astype(jnp.int32)
    i2 = i_id.reshape(BATCH // CHUNK, CHUNK).astype(jnp.int32)
    mean16 = jnp.broadcast_to(mean, (L,))
    return f(u2, i2, user_emb, user_bias.reshape(-1),
             item_emb, item_bias.reshape(-1), mean16)
